# baseline (speedup 1.0000x reference)
import functools

import jax
import jax.numpy as jnp
from jax import lax
from jax.experimental import pallas as pl
from jax.experimental.pallas import tpu as pltpu
from jax.experimental.pallas import tpu_sc as plsc

B = 16384
D = 64
NC = 2
NS = 16
NW = NC * NS
BPW = B // NW
CHUNK = 128
NCH = BPW // CHUNK
LANES = 16
NBUF = 2

_MESH = plsc.VectorSubcoreMesh(core_axis_name="c", subcore_axis_name="s")


@functools.partial(
    pl.kernel,
    mesh=_MESH,
    out_type=jax.ShapeDtypeStruct((B, 6 * D), jnp.float32),
    scratch_types=[
        pltpu.VMEM((6, BPW), jnp.int32),
        pltpu.VMEM((NCH, CHUNK), jnp.int32),
        pltpu.VMEM((NCH, CHUNK), jnp.int32),
        pltpu.VMEM((NCH, CHUNK), jnp.int32),
        pltpu.VMEM((CHUNK, 6 * D), jnp.float32),
        pltpu.VMEM((CHUNK, 6 * D), jnp.float32),
        pltpu.SemaphoreType.DMA,
        pltpu.SemaphoreType.DMA,
        pltpu.SemaphoreType.DMA,
        pltpu.SemaphoreType.DMA,
    ],
)
def _sc_embed(tall, call, out,
              codes, idx12, idx34, idx56, b0, b1,
              sg0, sg1, sw0, sw1):
    wid = lax.axis_index("s") * NC + lax.axis_index("c")
    base = wid * BPW

    pltpu.async_copy(call.at[:, pl.ds(base, BPW)], codes, sg0).wait()

    tabs = (tall, tall, tall)
    idxs = (idx12, idx34, idx56)
    bufs = (b0, b1)
    gsem = (sg0, sg1)
    wsem = (sw0, sw1)

    def compute_idx(c):
        for k in range(CHUNK // LANES):
            s = c * CHUNK + k * LANES
            sl = pl.ds(s, LANES)
            ksl = pl.ds(k * LANES, LANES)
            idx12[c, ksl] = codes[0, sl] * 11 + codes[1, sl]
            idx34[c, ksl] = codes[2, sl] * 12 + codes[3, sl] + 132
            idx56[c, ksl] = codes[4, sl] * 24 + codes[5, sl] + 588

    def issue_gathers(c, s):
        return [pltpu.async_copy(tabs[p].at[idxs[p].at[c]],
                                 bufs[s].at[:, pl.ds(p * 2 * D, 2 * D)],
                                 gsem[s])
                for p in range(3)]

    def issue_write(c, s):
        return pltpu.async_copy(
            bufs[s], out.at[pl.ds(base + c * CHUNK, CHUNK)], wsem[s])

    pend_g = [None] * NBUF
    pend_w = [None] * NBUF
    for c in range(NBUF - 1):
        compute_idx(c)
        pend_g[c % NBUF] = issue_gathers(c, c % NBUF)
    for c in range(NBUF - 1, NCH):
        compute_idx(c)
    for c in range(NCH):
        s = c % NBUF
        ahead = c + NBUF - 1
        if ahead < NCH:
            s2 = ahead % NBUF
            if pend_w[s2] is not None:
                pend_w[s2].wait()
                pend_w[s2] = None
            pend_g[s2] = issue_gathers(ahead, s2)
        for g in pend_g[s]:
            g.wait()
        pend_w[s] = issue_write(c, s)
    for s in range(NBUF):
        if pend_w[s] is not None:
            pend_w[s].wait()


def _halfrow_ids():
    ids = []
    for (va, oa), (vb, ob) in (((12, 0), (11, 12)), ((38, 23), (12, 61)),
                               ((7, 73), (24, 85))):
        for a in range(va):
            for b in range(vb):
                ids.append(oa + a)
                ids.append(ob + b)
    return tuple(ids)


_IDS = _halfrow_ids()


def kernel(code_holiday, code_weather, code_weather_detail, code_month,
           code_dayofweek, code_hour, W_holiday, W_weather, W_weather_detail,
           W_month, W_dayofweek, W_hour):
    wcat = jnp.concatenate([W_holiday, W_weather, W_weather_detail,
                            W_month, W_dayofweek, W_hour], axis=0)
    tall = jnp.take(wcat, jnp.asarray(_IDS, dtype=jnp.int32),
                    axis=0).reshape(756, 2 * D)
    call = jnp.stack([c.astype(jnp.int32) for c in (
        code_holiday, code_weather, code_weather_detail,
        code_month, code_dayofweek, code_hour)])
    return _sc_embed(tall, call)


# speedup vs baseline: 1.0933x; 1.0933x over previous
import functools

import jax
import jax.numpy as jnp
from jax import lax
from jax.experimental import pallas as pl
from jax.experimental.pallas import tpu as pltpu
from jax.experimental.pallas import tpu_sc as plsc

B = 16384
D = 64
NC = 2
NS = 16
NW = NC * NS
BPW = B // NW
CHUNK = 128
NCH = BPW // CHUNK
LANES = 16
NBUF = 2

_MESH = plsc.VectorSubcoreMesh(core_axis_name="c", subcore_axis_name="s")


@functools.partial(
    pl.kernel,
    mesh=_MESH,
    out_type=jax.ShapeDtypeStruct((B, 6 * D), jnp.float32),
    scratch_types=[
        pltpu.VMEM((6, BPW), jnp.int32),
        pltpu.VMEM((NCH, CHUNK), jnp.int32),
        pltpu.VMEM((NCH, CHUNK), jnp.int32),
        pltpu.VMEM((NCH, CHUNK), jnp.int32),
        pltpu.VMEM((CHUNK, 6 * D), jnp.float32),
        pltpu.VMEM((CHUNK, 6 * D), jnp.float32),
        pltpu.SemaphoreType.DMA,
        pltpu.SemaphoreType.DMA,
        pltpu.SemaphoreType.DMA,
        pltpu.SemaphoreType.DMA,
    ],
)
def _sc_embed(t12, t34, t56, c1, c2, c3, c4, c5, c6, out,
              codes, idx12, idx34, idx56, b0, b1,
              sg0, sg1, sw0, sw1):
    wid = lax.axis_index("s") * NC + lax.axis_index("c")
    base = wid * BPW

    cps = [pltpu.async_copy(src.at[pl.ds(base, BPW)], codes.at[i], sg0)
           for i, src in enumerate((c1, c2, c3, c4, c5, c6))]
    for cp in cps:
        cp.wait()

    tabs = (t12, t34, t56)
    idxs = (idx12, idx34, idx56)
    bufs = (b0, b1)
    gsem = (sg0, sg1)
    wsem = (sw0, sw1)

    def compute_idx(c):
        for k in range(CHUNK // LANES):
            s = c * CHUNK + k * LANES
            sl = pl.ds(s, LANES)
            ksl = pl.ds(k * LANES, LANES)
            idx12[c, ksl] = codes[0, sl] * 11 + codes[1, sl]
            idx34[c, ksl] = codes[2, sl] * 12 + codes[3, sl]
            idx56[c, ksl] = codes[4, sl] * 24 + codes[5, sl]

    def issue_gathers(c, s):
        return [pltpu.async_copy(tabs[p].at[idxs[p].at[c]],
                                 bufs[s].at[:, pl.ds(p * 2 * D, 2 * D)],
                                 gsem[s])
                for p in range(3)]

    def issue_write(c, s):
        return pltpu.async_copy(
            bufs[s], out.at[pl.ds(base + c * CHUNK, CHUNK)], wsem[s])

    pend_g = [None] * NBUF
    pend_w = [None] * NBUF
    for c in range(NBUF - 1):
        compute_idx(c)
        pend_g[c % NBUF] = issue_gathers(c, c % NBUF)
    for c in range(NBUF - 1, NCH):
        compute_idx(c)
    for c in range(NCH):
        s = c % NBUF
        ahead = c + NBUF - 1
        if ahead < NCH:
            s2 = ahead % NBUF
            if pend_w[s2] is not None:
                pend_w[s2].wait()
                pend_w[s2] = None
            pend_g[s2] = issue_gathers(ahead, s2)
        for g in pend_g[s]:
            g.wait()
        pend_w[s] = issue_write(c, s)
    for s in range(NBUF):
        if pend_w[s] is not None:
            pend_w[s].wait()


def kernel(code_holiday, code_weather, code_weather_detail, code_month,
           code_dayofweek, code_hour, W_holiday, W_weather, W_weather_detail,
           W_month, W_dayofweek, W_hour):
    def fuse(A, B):
        va, vb = A.shape[0], B.shape[0]
        left = jnp.pad(A, ((0, 0), (0, D)))[:, None, :]
        right = jnp.pad(B, ((0, 0), (D, 0)))[None, :, :]
        return (left + right).reshape(va * vb, 2 * D)

    t12 = fuse(W_holiday, W_weather)
    t34 = fuse(W_weather_detail, W_month)
    t56 = fuse(W_dayofweek, W_hour)

    codes = [c.astype(jnp.int32) for c in (
        code_holiday, code_weather, code_weather_detail,
        code_month, code_dayofweek, code_hour)]
    return _sc_embed(t12, t34, t56, *codes)


# speedup vs baseline: 1.1776x; 1.0771x over previous
import functools

import jax
import jax.numpy as jnp
from jax import lax
from jax.experimental import pallas as pl
from jax.experimental.pallas import tpu as pltpu
from jax.experimental.pallas import tpu_sc as plsc

B = 16384
D = 64
S = 8192
NC = 2
NS = 16
NW = NC * NS
BPW = S // NW
CHUNK = 128
NCH = BPW // CHUNK
LANES = 16
NBUF = 2

_MESH = plsc.VectorSubcoreMesh(core_axis_name="c", subcore_axis_name="s")


@functools.partial(
    pl.kernel,
    mesh=_MESH,
    out_type=jax.ShapeDtypeStruct((B, 6 * D), jnp.float32),
    scratch_types=[
        pltpu.VMEM((6, BPW), jnp.int32),
        pltpu.VMEM((NCH, CHUNK), jnp.int32),
        pltpu.VMEM((NCH, CHUNK), jnp.int32),
        pltpu.VMEM((NCH, CHUNK), jnp.int32),
        pltpu.VMEM((CHUNK, 6 * D), jnp.float32),
        pltpu.VMEM((CHUNK, 6 * D), jnp.float32),
        pltpu.SemaphoreType.DMA,
        pltpu.SemaphoreType.DMA,
        pltpu.SemaphoreType.DMA,
        pltpu.SemaphoreType.DMA,
    ],
)
def _sc_embed(t12, t34, t56, c1, c2, c3, c4, c5, c6, out,
              codes, idx12, idx34, idx56, b0, b1,
              sg0, sg1, sw0, sw1):
    wid = lax.axis_index("s") * NC + lax.axis_index("c")
    base = wid * BPW

    cps = [pltpu.async_copy(src.at[pl.ds(base, BPW)], codes.at[i], sg0)
           for i, src in enumerate((c1, c2, c3, c4, c5, c6))]
    for cp in cps:
        cp.wait()

    tabs = (t12, t34, t56)
    idxs = (idx12, idx34, idx56)
    bufs = (b0, b1)
    gsem = (sg0, sg1)
    wsem = (sw0, sw1)

    def compute_idx(c):
        for k in range(CHUNK // LANES):
            s = c * CHUNK + k * LANES
            sl = pl.ds(s, LANES)
            ksl = pl.ds(k * LANES, LANES)
            idx12[c, ksl] = codes[0, sl] * 11 + codes[1, sl]
            idx34[c, ksl] = codes[2, sl] * 12 + codes[3, sl]
            idx56[c, ksl] = codes[4, sl] * 24 + codes[5, sl]

    def issue_gathers(c, s):
        return [pltpu.async_copy(tabs[p].at[idxs[p].at[c]],
                                 bufs[s].at[:, pl.ds(p * 2 * D, 2 * D)],
                                 gsem[s])
                for p in range(3)]

    def issue_write(c, s):
        return pltpu.async_copy(
            bufs[s], out.at[pl.ds(base + c * CHUNK, CHUNK)], wsem[s])

    pend_g = [None] * NBUF
    pend_w = [None] * NBUF
    for c in range(NBUF - 1):
        compute_idx(c)
        pend_g[c % NBUF] = issue_gathers(c, c % NBUF)
    for c in range(NBUF - 1, NCH):
        compute_idx(c)
    for c in range(NCH):
        s = c % NBUF
        ahead = c + NBUF - 1
        if ahead < NCH:
            s2 = ahead % NBUF
            if pend_w[s2] is not None:
                pend_w[s2].wait()
                pend_w[s2] = None
            pend_g[s2] = issue_gathers(ahead, s2)
        for g in pend_g[s]:
            g.wait()
        pend_w[s] = issue_write(c, s)
    for s in range(NBUF):
        if pend_w[s] is not None:
            pend_w[s].wait()


TCBLK = 512
_NBLK = B // TCBLK


def _tc_body(o_alias, c1, c2, c3, c4, c5, c6, w1, w2, w3, w4, w5, w6, out):
    del o_alias
    tabs = (w1, w2, w3, w4, w5, w6)
    cods = (c1, c2, c3, c4, c5, c6)
    parts = []
    for t in range(6):
        v = tabs[t].shape[0]
        code = cods[t][0, 0, :]
        onehot = (code[:, None] == lax.broadcasted_iota(jnp.int32, (TCBLK, v), 1)
                  ).astype(jnp.float32)
        parts.append(jnp.dot(onehot, tabs[t][:, :],
                             preferred_element_type=jnp.float32))
    out[:, :] = jnp.concatenate(parts, axis=1)


def _make_tc_tail():
    grid = (B - S) // TCBLK
    off = S // TCBLK
    cspec = pl.BlockSpec((1, 1, TCBLK), lambda j: (j + off, 0, 0))
    return pl.pallas_call(
        _tc_body,
        grid=(grid,),
        in_specs=[pl.BlockSpec(memory_space=pltpu.MemorySpace.HBM)]
                 + [cspec] * 6
                 + [pl.BlockSpec((12, D), lambda j: (0, 0)),
                    pl.BlockSpec((11, D), lambda j: (0, 0)),
                    pl.BlockSpec((38, D), lambda j: (0, 0)),
                    pl.BlockSpec((12, D), lambda j: (0, 0)),
                    pl.BlockSpec((7, D), lambda j: (0, 0)),
                    pl.BlockSpec((24, D), lambda j: (0, 0))],
        out_specs=pl.BlockSpec((TCBLK, 6 * D), lambda j: (j + off, 0)),
        out_shape=jax.ShapeDtypeStruct((B, 6 * D), jnp.float32),
        input_output_aliases={0: 0},
    )


def kernel(code_holiday, code_weather, code_weather_detail, code_month,
           code_dayofweek, code_hour, W_holiday, W_weather, W_weather_detail,
           W_month, W_dayofweek, W_hour):
    t12 = jnp.concatenate([
        jnp.broadcast_to(W_holiday[:, None, :], (12, 11, D)),
        jnp.broadcast_to(W_weather[None, :, :], (12, 11, D)),
    ], axis=2).reshape(12 * 11, 2 * D)
    t34 = jnp.concatenate([
        jnp.broadcast_to(W_weather_detail[:, None, :], (38, 12, D)),
        jnp.broadcast_to(W_month[None, :, :], (38, 12, D)),
    ], axis=2).reshape(38 * 12, 2 * D)
    t56 = jnp.concatenate([
        jnp.broadcast_to(W_dayofweek[:, None, :], (7, 24, D)),
        jnp.broadcast_to(W_hour[None, :, :], (7, 24, D)),
    ], axis=2).reshape(7 * 24, 2 * D)

    codes = [c.astype(jnp.int32) for c in (
        code_holiday, code_weather, code_weather_detail,
        code_month, code_dayofweek, code_hour)]
    out_sc = _sc_embed(t12, t34, t56, *codes)
    codes3d = [c.reshape(_NBLK, 1, TCBLK) for c in codes]
    tables = (W_holiday, W_weather, W_weather_detail,
              W_month, W_dayofweek, W_hour)
    return _make_tc_tail()(out_sc, *codes3d, *tables)


# speedup vs baseline: 1.2091x; 1.0268x over previous
import functools

import jax
import jax.numpy as jnp
from jax import lax
from jax.experimental import pallas as pl
from jax.experimental.pallas import tpu as pltpu
from jax.experimental.pallas import tpu_sc as plsc

B = 16384
D = 64
S = 4096
NC = 2
NS = 16
NW = NC * NS
BPW = S // NW
CHUNK = 128
NCH = BPW // CHUNK
LANES = 16
NBUF = 2

_MESH = plsc.VectorSubcoreMesh(core_axis_name="c", subcore_axis_name="s")


@functools.partial(
    pl.kernel,
    mesh=_MESH,
    out_type=jax.ShapeDtypeStruct((B, 6 * D), jnp.float32),
    scratch_types=[
        pltpu.VMEM((6, BPW), jnp.int32),
        pltpu.VMEM((NCH, CHUNK), jnp.int32),
        pltpu.VMEM((NCH, CHUNK), jnp.int32),
        pltpu.VMEM((NCH, CHUNK), jnp.int32),
        pltpu.VMEM((CHUNK, 6 * D), jnp.float32),
        pltpu.VMEM((CHUNK, 6 * D), jnp.float32),
        pltpu.SemaphoreType.DMA,
        pltpu.SemaphoreType.DMA,
        pltpu.SemaphoreType.DMA,
        pltpu.SemaphoreType.DMA,
    ],
)
def _sc_embed(t12, t34, t56, c1, c2, c3, c4, c5, c6, out,
              codes, idx12, idx34, idx56, b0, b1,
              sg0, sg1, sw0, sw1):
    wid = lax.axis_index("s") * NC + lax.axis_index("c")
    base = wid * BPW

    cps = [pltpu.async_copy(src.at[pl.ds(base, BPW)], codes.at[i], sg0)
           for i, src in enumerate((c1, c2, c3, c4, c5, c6))]
    for cp in cps:
        cp.wait()

    tabs = (t12, t34, t56)
    idxs = (idx12, idx34, idx56)
    bufs = (b0, b1)
    gsem = (sg0, sg1)
    wsem = (sw0, sw1)

    def compute_idx(c):
        for k in range(CHUNK // LANES):
            s = c * CHUNK + k * LANES
            sl = pl.ds(s, LANES)
            ksl = pl.ds(k * LANES, LANES)
            idx12[c, ksl] = codes[0, sl] * 11 + codes[1, sl]
            idx34[c, ksl] = codes[2, sl] * 12 + codes[3, sl]
            idx56[c, ksl] = codes[4, sl] * 24 + codes[5, sl]

    def issue_gathers(c, s):
        return [pltpu.async_copy(tabs[p].at[idxs[p].at[c]],
                                 bufs[s].at[:, pl.ds(p * 2 * D, 2 * D)],
                                 gsem[s])
                for p in range(3)]

    def issue_write(c, s):
        return pltpu.async_copy(
            bufs[s], out.at[pl.ds(base + c * CHUNK, CHUNK)], wsem[s])

    pend_g = [None] * NBUF
    pend_w = [None] * NBUF
    for c in range(NBUF - 1):
        compute_idx(c)
        pend_g[c % NBUF] = issue_gathers(c, c % NBUF)
    for c in range(NBUF - 1, NCH):
        compute_idx(c)
    for c in range(NCH):
        s = c % NBUF
        ahead = c + NBUF - 1
        if ahead < NCH:
            s2 = ahead % NBUF
            if pend_w[s2] is not None:
                pend_w[s2].wait()
                pend_w[s2] = None
            pend_g[s2] = issue_gathers(ahead, s2)
        for g in pend_g[s]:
            g.wait()
        pend_w[s] = issue_write(c, s)
    for s in range(NBUF):
        if pend_w[s] is not None:
            pend_w[s].wait()


TCBLK = 512
_NBLK = B // TCBLK


def _tc_body(o_alias, c1, c2, c3, c4, c5, c6, w1, w2, w3, w4, w5, w6, out):
    del o_alias
    tabs = (w1, w2, w3, w4, w5, w6)
    cods = (c1, c2, c3, c4, c5, c6)
    parts = []
    for t in range(6):
        v = tabs[t].shape[0]
        code = cods[t][0, 0, :]
        onehot = (code[:, None] == lax.broadcasted_iota(jnp.int32, (TCBLK, v), 1)
                  ).astype(jnp.float32)
        parts.append(jnp.dot(onehot, tabs[t][:, :],
                             preferred_element_type=jnp.float32))
    out[:, :] = jnp.concatenate(parts, axis=1)


def _make_tc_tail():
    grid = (B - S) // TCBLK
    off = S // TCBLK
    cspec = pl.BlockSpec((1, 1, TCBLK), lambda j: (j + off, 0, 0))
    return pl.pallas_call(
        _tc_body,
        grid=(grid,),
        in_specs=[pl.BlockSpec(memory_space=pltpu.MemorySpace.HBM)]
                 + [cspec] * 6
                 + [pl.BlockSpec((12, D), lambda j: (0, 0)),
                    pl.BlockSpec((11, D), lambda j: (0, 0)),
                    pl.BlockSpec((38, D), lambda j: (0, 0)),
                    pl.BlockSpec((12, D), lambda j: (0, 0)),
                    pl.BlockSpec((7, D), lambda j: (0, 0)),
                    pl.BlockSpec((24, D), lambda j: (0, 0))],
        out_specs=pl.BlockSpec((TCBLK, 6 * D), lambda j: (j + off, 0)),
        out_shape=jax.ShapeDtypeStruct((B, 6 * D), jnp.float32),
        input_output_aliases={0: 0},
    )


def kernel(code_holiday, code_weather, code_weather_detail, code_month,
           code_dayofweek, code_hour, W_holiday, W_weather, W_weather_detail,
           W_month, W_dayofweek, W_hour):
    t12 = jnp.concatenate([
        jnp.broadcast_to(W_holiday[:, None, :], (12, 11, D)),
        jnp.broadcast_to(W_weather[None, :, :], (12, 11, D)),
    ], axis=2).reshape(12 * 11, 2 * D)
    t34 = jnp.concatenate([
        jnp.broadcast_to(W_weather_detail[:, None, :], (38, 12, D)),
        jnp.broadcast_to(W_month[None, :, :], (38, 12, D)),
    ], axis=2).reshape(38 * 12, 2 * D)
    t56 = jnp.concatenate([
        jnp.broadcast_to(W_dayofweek[:, None, :], (7, 24, D)),
        jnp.broadcast_to(W_hour[None, :, :], (7, 24, D)),
    ], axis=2).reshape(7 * 24, 2 * D)

    codes = [c.astype(jnp.int32) for c in (
        code_holiday, code_weather, code_weather_detail,
        code_month, code_dayofweek, code_hour)]
    out_sc = _sc_embed(t12, t34, t56, *codes)
    codes3d = [c.reshape(_NBLK, 1, TCBLK) for c in codes]
    tables = (W_holiday, W_weather, W_weather_detail,
              W_month, W_dayofweek, W_hour)
    return _make_tc_tail()(out_sc, *codes3d, *tables)
